# baseline (device time: 176386 ns/iter reference)
import jax
import jax.numpy as jnp
from jax import lax
from jax.experimental import pallas as pl
from jax.experimental.pallas import tpu as pltpu

N_DEV = 4
HQ = 8
H_ALL = 32
DH = 128
BLK = 64
N_RES = 4
F32 = jnp.float32
BF16 = jnp.bfloat16
SCALE = 0.08838834764831843
DEPTH = 2

_CompilerParams = getattr(pltpu, "CompilerParams", None) or getattr(
    pltpu, "TPUCompilerParams"
)


def kernel(x, Wq, K_ext, V_ext, Wo):
    _, SQ, D_MODEL = x.shape
    SKV = K_ext.shape[1]
    HD = HQ * DH
    n_qb = SQ // BLK
    m_per_res = n_qb // N_RES

    def body(x_ref, wq_ref, k_ref, v_ref, wo_ref, out_ref,
             xv, wv, qv, outv, stage, k_bf, v_bf, k_all, v_all,
             p_send, p_recv, lsq,
             k_send_s, k_recv_s, v_send_s, v_recv_s,
             p_send_s, p_recv_s, conv_s, loc_s):
        my = lax.axis_index("i")

        cx = pltpu.make_async_copy(x_ref.at[0], xv, loc_s.at[2])
        cwq = pltpu.make_async_copy(wq_ref, wv, loc_s.at[3])
        cx.start()
        cwq.start()

        bsem = pltpu.get_barrier_semaphore()
        for d in range(1, N_DEV):
            peer = lax.rem(my + d, N_DEV)
            pl.semaphore_signal(bsem, inc=1, device_id=(peer,),
                                device_id_type=pl.DeviceIdType.MESH)
        pl.semaphore_wait(bsem, N_DEV - 1)

        pieces = [(k_ref, k_bf, h) for h in range(H_ALL)] + \
                 [(v_ref, v_bf, h) for h in range(H_ALL)]

        def piece_copy(i):
            src, _, h = pieces[i]
            c = pltpu.make_async_copy(
                src.at[0, :, h, :], stage.at[i % DEPTH], conv_s.at[i % DEPTH])
            c.start()
            return c

        kv_descs_by_d = {1: [], 2: [], 3: []}
        loc_descs = []

        def start_sends(src_bf, allbuf, ss, rs, loc_sem, dlist):
            for d in dlist:
                peer = lax.rem(my + d, N_DEV)
                for h in range(HQ):
                    c = pltpu.make_async_remote_copy(
                        src_ref=src_bf.at[peer * HQ + h],
                        dst_ref=allbuf.at[d - 1, h],
                        send_sem=ss.at[d - 1],
                        recv_sem=rs.at[d - 1],
                        device_id=(peer,),
                        device_id_type=pl.DeviceIdType.MESH,
                    )
                    c.start()
                    kv_descs_by_d[d].append(c)
            if loc_sem is not None:
                for h in range(HQ):
                    c = pltpu.make_async_copy(
                        src_bf.at[my * HQ + h], allbuf.at[N_DEV - 1, h],
                        loc_sem)
                    c.start()
                    loc_descs.append(c)

        inflight = [piece_copy(i) for i in range(DEPTH)]
        for i in range(len(pieces)):
            inflight[i % DEPTH].wait()
            _, dst_bf, h = pieces[i]
            dst_bf[h] = stage[i % DEPTH].astype(BF16)
            if i + DEPTH < len(pieces):
                inflight[i % DEPTH] = piece_copy(i + DEPTH)
            if i == H_ALL - 1:
                start_sends(k_bf, k_all, k_send_s, k_recv_s, loc_s.at[0],
                            [1, 3])
        start_sends(v_bf, v_all, v_send_s, v_recv_s, loc_s.at[1], [1, 3])

        cx.wait()
        cwq.wait()
        qv[...] = jnp.dot(xv[...], wv[...],
                          preferred_element_type=F32).astype(BF16)
        cwo = pltpu.make_async_copy(wo_ref, wv, loc_s.at[4])
        cwo.start()

        for d in (1, 3):
            for c in kv_descs_by_d[d]:
                c.wait_send()
        start_sends(k_bf, k_all, k_send_s, k_recv_s, None, [2])
        start_sends(v_bf, v_all, v_send_s, v_recv_s, None, [2])

        for c in loc_descs:
            c.wait()

        cwo.wait()
        rows_of = [[(m * N_RES + r) * BLK for m in range(m_per_res)]
                   for r in range(N_RES)]
        p_descs = []

        def attn_pass(slots, first):
            for r in range(N_RES):
                rows = rows_of[r]
                rr = r * m_per_res * BLK
                sums = []
                for h in range(HQ):
                    hc = slice(h * DH, (h + 1) * DH)
                    qr = jnp.concatenate(
                        [qv[o:o + BLK, hc] for o in rows], axis=0)
                    kr = jnp.concatenate(
                        [k_all[c, h, o:o + BLK, :]
                         for c in slots for o in rows], axis=0)
                    vr = jnp.concatenate(
                        [v_all[c, h, o:o + BLK, :]
                         for c in slots for o in rows], axis=0)
                    s = lax.dot_general(
                        qr, kr, (((1,), (1,)), ((), ())),
                        preferred_element_type=F32) * SCALE
                    e = jnp.exp(s)
                    pv = jnp.dot(e.astype(BF16), vr,
                                 preferred_element_type=F32)
                    xv[rr:rr + 256, hc] = (
                        pv if first else xv[rr:rr + 256, hc] + pv)
                    sums.append(jnp.sum(e, axis=1, keepdims=True))
                ls = jnp.concatenate(sums, axis=1)
                lsq[r] = ls if first else lsq[r] + ls

        for d in (1, 3):
            for c in kv_descs_by_d[d]:
                c.wait_recv()
        attn_pass([N_DEV - 1, 0, 2], True)

        for c in kv_descs_by_d[2]:
            c.wait_recv()
        attn_pass([1], False)

        for r in range(N_RES):
            rr = r * m_per_res * BLK
            lr = lsq[r]
            ctx_r = jnp.concatenate(
                [xv[rr:rr + 256, h * DH:(h + 1) * DH] / lr[:, h:h + 1]
                 for h in range(HQ)],
                axis=1)
            out_r = jnp.dot(ctx_r, wv[...], preferred_element_type=F32)
            outv[rr:rr + m_per_res * BLK, :] = out_r
            p_send[r] = out_r.astype(BF16)
            for d in range(1, N_DEV):
                peer = lax.rem(my + d, N_DEV)
                c = pltpu.make_async_remote_copy(
                    src_ref=p_send.at[r],
                    dst_ref=p_recv.at[d - 1, r],
                    send_sem=p_send_s.at[d - 1],
                    recv_sem=p_recv_s.at[d - 1],
                    device_id=(peer,),
                    device_id_type=pl.DeviceIdType.MESH,
                )
                c.start()
                p_descs.append(c)

        for c in kv_descs_by_d[2]:
            c.wait_send()
        for c in p_descs:
            c.wait_recv()
        for c in p_descs:
            c.wait_send()
        acc = outv[...]
        for d in range(N_DEV - 1):
            acc = acc + p_recv[d].astype(F32).reshape(SQ, D_MODEL)
        outv[...] = acc

        out_descs = []
        for r in range(N_RES):
            for m in range(m_per_res):
                c = pltpu.make_async_copy(
                    outv.at[pl.ds((r * m_per_res + m) * BLK, BLK), :],
                    out_ref.at[0, pl.ds((m * N_RES + r) * BLK, BLK), :],
                    loc_s.at[5])
                c.start()
                out_descs.append(c)
        for c in out_descs:
            c.wait()

    return pl.pallas_call(
        body,
        out_shape=jax.ShapeDtypeStruct((1, SQ, D_MODEL), F32),
        in_specs=[pl.BlockSpec(memory_space=pl.ANY)] * 5,
        out_specs=pl.BlockSpec(memory_space=pl.ANY),
        scratch_shapes=[
            pltpu.VMEM((SQ, D_MODEL), F32),
            pltpu.VMEM((D_MODEL, HD), F32),
            pltpu.VMEM((SQ, HD), BF16),
            pltpu.VMEM((SQ, D_MODEL), F32),
            pltpu.VMEM((DEPTH, SKV, DH), F32),
            pltpu.VMEM((H_ALL, SKV, DH), BF16),
            pltpu.VMEM((H_ALL, SKV, DH), BF16),
            pltpu.VMEM((N_DEV, HQ, SKV, DH), BF16),
            pltpu.VMEM((N_DEV, HQ, SKV, DH), BF16),
            pltpu.VMEM((N_RES, SQ // N_RES, D_MODEL), BF16),
            pltpu.VMEM((N_DEV - 1, N_RES, SQ // N_RES, D_MODEL), BF16),
            pltpu.VMEM((N_RES, SQ // N_RES, HQ), F32),
            pltpu.SemaphoreType.DMA((N_DEV - 1,)),
            pltpu.SemaphoreType.DMA((N_DEV - 1,)),
            pltpu.SemaphoreType.DMA((N_DEV - 1,)),
            pltpu.SemaphoreType.DMA((N_DEV - 1,)),
            pltpu.SemaphoreType.DMA((N_DEV - 1,)),
            pltpu.SemaphoreType.DMA((N_DEV - 1,)),
            pltpu.SemaphoreType.DMA((DEPTH,)),
            pltpu.SemaphoreType.DMA((6,)),
        ],
        compiler_params=_CompilerParams(
            collective_id=0, vmem_limit_bytes=63 * 1024 * 1024),
    )(x, Wq, K_ext, V_ext, Wo)


# device time: 167060 ns/iter; 1.0558x vs baseline; 1.0558x over previous
import jax
import jax.numpy as jnp
from jax import lax
from jax.experimental import pallas as pl
from jax.experimental.pallas import tpu as pltpu

N_DEV = 4
HQ = 8
H_ALL = 32
DH = 128
BLK = 64
N_RES = 4
F32 = jnp.float32
BF16 = jnp.bfloat16
SCALE = 0.08838834764831843
DEPTH = 4

_CompilerParams = getattr(pltpu, "CompilerParams", None) or getattr(
    pltpu, "TPUCompilerParams"
)


def kernel(x, Wq, K_ext, V_ext, Wo):
    _, SQ, D_MODEL = x.shape
    SKV = K_ext.shape[1]
    HD = HQ * DH
    n_qb = SQ // BLK
    m_per_res = n_qb // N_RES

    def body(x_ref, wq_ref, k_ref, v_ref, wo_ref, out_ref,
             xv, wv, qv, outv, stage, k_bf, v_bf, k_all, v_all,
             p_send, p_recv,
             k_send_s, k_recv_s, v_send_s, v_recv_s,
             p_send_s, p_recv_s, conv_s, loc_s):
        my = lax.axis_index("i")

        cx = pltpu.make_async_copy(x_ref.at[0], xv, loc_s.at[2])
        cwq = pltpu.make_async_copy(wq_ref, wv, loc_s.at[3])
        cx.start()
        cwq.start()

        bsem = pltpu.get_barrier_semaphore()
        for d in range(1, N_DEV):
            peer = lax.rem(my + d, N_DEV)
            pl.semaphore_signal(bsem, inc=1, device_id=(peer,),
                                device_id_type=pl.DeviceIdType.MESH)
        pl.semaphore_wait(bsem, N_DEV - 1)

        pieces = [(k_ref, k_bf, h) for h in range(H_ALL)] + \
                 [(v_ref, v_bf, h) for h in range(H_ALL)]

        def piece_copy(i):
            src, _, h = pieces[i]
            c = pltpu.make_async_copy(
                src.at[0, :, h, :], stage.at[i % DEPTH], conv_s.at[i % DEPTH])
            c.start()
            return c

        kv_descs = []
        loc_descs = []

        def start_sends(src_bf, allbuf, ss, rs, loc_sem):
            for d in range(1, N_DEV):
                peer = lax.rem(my + d, N_DEV)
                for h in range(HQ):
                    c = pltpu.make_async_remote_copy(
                        src_ref=src_bf.at[peer * HQ + h],
                        dst_ref=allbuf.at[d - 1, h],
                        send_sem=ss.at[d - 1],
                        recv_sem=rs.at[d - 1],
                        device_id=(peer,),
                        device_id_type=pl.DeviceIdType.MESH,
                    )
                    c.start()
                    kv_descs.append(c)
            for h in range(HQ):
                c = pltpu.make_async_copy(
                    src_bf.at[my * HQ + h], allbuf.at[N_DEV - 1, h], loc_sem)
                c.start()
                loc_descs.append(c)

        inflight = [piece_copy(i) for i in range(DEPTH)]
        for i in range(len(pieces)):
            inflight[i % DEPTH].wait()
            _, dst_bf, h = pieces[i]
            dst_bf[h] = stage[i % DEPTH].astype(BF16)
            if i + DEPTH < len(pieces):
                inflight[i % DEPTH] = piece_copy(i + DEPTH)
            if i == H_ALL - 1:
                start_sends(k_bf, k_all, k_send_s, k_recv_s, loc_s.at[0])
        start_sends(v_bf, v_all, v_send_s, v_recv_s, loc_s.at[1])

        cx.wait()
        cwq.wait()
        qv[...] = jnp.dot(xv[...], wv[...],
                          preferred_element_type=F32).astype(BF16)
        cwo = pltpu.make_async_copy(wo_ref, wv, loc_s.at[4])
        cwo.start()

        for c in loc_descs:
            c.wait()
        for c in kv_descs:
            c.wait_recv()
        for c in kv_descs:
            c.wait_send()

        cwo.wait()
        p_descs = []
        for r in range(N_RES):
            rows = [(m * N_RES + r) * BLK for m in range(m_per_res)]
            ctx_cols = []
            for h in range(HQ):
                hc = slice(h * DH, (h + 1) * DH)
                qr = jnp.concatenate(
                    [qv[o:o + BLK, hc] for o in rows], axis=0)
                kr = jnp.concatenate(
                    [k_all[c, h, o:o + BLK, :]
                     for c in range(N_DEV) for o in rows], axis=0)
                vr = jnp.concatenate(
                    [v_all[c, h, o:o + BLK, :]
                     for c in range(N_DEV) for o in rows], axis=0)
                s = lax.dot_general(
                    qr, kr, (((1,), (1,)), ((), ())),
                    preferred_element_type=F32) * SCALE
                mx = jnp.max(s, axis=1, keepdims=True)
                e = jnp.exp(s - mx)
                p = (e / jnp.sum(e, axis=1, keepdims=True)).astype(BF16)
                ctx_cols.append(
                    jnp.dot(p, vr, preferred_element_type=F32))
            ctx_r = jnp.concatenate(ctx_cols, axis=1)
            out_r = jnp.dot(ctx_r, wv[...], preferred_element_type=F32)
            rr = r * m_per_res * BLK
            outv[rr:rr + m_per_res * BLK, :] = out_r
            p_send[r] = out_r.astype(BF16)
            for d in range(1, N_DEV):
                peer = lax.rem(my + d, N_DEV)
                c = pltpu.make_async_remote_copy(
                    src_ref=p_send.at[r],
                    dst_ref=p_recv.at[d - 1, r],
                    send_sem=p_send_s.at[d - 1],
                    recv_sem=p_recv_s.at[d - 1],
                    device_id=(peer,),
                    device_id_type=pl.DeviceIdType.MESH,
                )
                c.start()
                p_descs.append(c)

        for c in p_descs:
            c.wait_recv()
        for c in p_descs:
            c.wait_send()
        acc = outv[...]
        for d in range(N_DEV - 1):
            acc = acc + p_recv[d].astype(F32).reshape(SQ, D_MODEL)
        outv[...] = acc

        out_descs = []
        for r in range(N_RES):
            for m in range(m_per_res):
                c = pltpu.make_async_copy(
                    outv.at[pl.ds((r * m_per_res + m) * BLK, BLK), :],
                    out_ref.at[0, pl.ds((m * N_RES + r) * BLK, BLK), :],
                    loc_s.at[5])
                c.start()
                out_descs.append(c)
        for c in out_descs:
            c.wait()

    return pl.pallas_call(
        body,
        out_shape=jax.ShapeDtypeStruct((1, SQ, D_MODEL), F32),
        in_specs=[pl.BlockSpec(memory_space=pl.ANY)] * 5,
        out_specs=pl.BlockSpec(memory_space=pl.ANY),
        scratch_shapes=[
            pltpu.VMEM((SQ, D_MODEL), F32),
            pltpu.VMEM((D_MODEL, HD), F32),
            pltpu.VMEM((SQ, HD), BF16),
            pltpu.VMEM((SQ, D_MODEL), F32),
            pltpu.VMEM((DEPTH, SKV, DH), F32),
            pltpu.VMEM((H_ALL, SKV, DH), BF16),
            pltpu.VMEM((H_ALL, SKV, DH), BF16),
            pltpu.VMEM((N_DEV, HQ, SKV, DH), BF16),
            pltpu.VMEM((N_DEV, HQ, SKV, DH), BF16),
            pltpu.VMEM((N_RES, SQ // N_RES, D_MODEL), BF16),
            pltpu.VMEM((N_DEV - 1, N_RES, SQ // N_RES, D_MODEL), BF16),
            pltpu.SemaphoreType.DMA((N_DEV - 1,)),
            pltpu.SemaphoreType.DMA((N_DEV - 1,)),
            pltpu.SemaphoreType.DMA((N_DEV - 1,)),
            pltpu.SemaphoreType.DMA((N_DEV - 1,)),
            pltpu.SemaphoreType.DMA((N_DEV - 1,)),
            pltpu.SemaphoreType.DMA((N_DEV - 1,)),
            pltpu.SemaphoreType.DMA((DEPTH,)),
            pltpu.SemaphoreType.DMA((6,)),
        ],
        compiler_params=_CompilerParams(
            collective_id=0, vmem_limit_bytes=63 * 1024 * 1024),
    )(x, Wq, K_ext, V_ext, Wo)
